# baseline (speedup 1.0000x reference)
import functools

import jax
import jax.numpy as jnp
from jax import lax
from jax.experimental import pallas as pl
from jax.experimental.pallas import tpu as pltpu
from jax.experimental.pallas import tpu_sc as plsc

BATCH = 4
SEQ = 4096
D = 768
NW = 32
ROWS_PER_W = SEQ // NW
STEP = 64
NSTEP = ROWS_PER_W // STEP


def _gather_kernel(idx_hbm, table_hbm, out_hbm, idx_v, rows_v, sem_g, sem_st):
    cid = lax.axis_index("c")
    sid = lax.axis_index("s")
    wid = sid * 2 + cid
    base = wid * ROWS_PER_W

    pltpu.sync_copy(idx_hbm.at[pl.ds(base, ROWS_PER_W)], idx_v)
    for s in range(NSTEP):
        pltpu.async_copy(
            table_hbm.at[idx_v.at[pl.ds(s * STEP, STEP)]],
            rows_v.at[s], sem_g.at[s])
    for s in range(NSTEP):
        pltpu.make_async_copy(
            table_hbm.at[idx_v.at[pl.ds(0, STEP)]],
            rows_v.at[s], sem_g.at[s]).wait()
        pltpu.async_copy(
            rows_v.at[s], out_hbm.at[pl.ds(base + s * STEP, STEP)],
            sem_st.at[s])
    for s in range(NSTEP):
        pltpu.make_async_copy(
            rows_v.at[s], out_hbm.at[pl.ds(0, STEP)], sem_st.at[s]).wait()


def _add_kernel(tmp_ref, pos_ref, o_ref):
    o_ref[...] = tmp_ref[...] + pos_ref[...]


def _sc_gather(idx, table):
    mesh = plsc.VectorSubcoreMesh(core_axis_name="c", subcore_axis_name="s")
    return functools.partial(
        pl.kernel,
        out_type=jax.ShapeDtypeStruct((SEQ, D), jnp.float32),
        mesh=mesh,
        scratch_types=[
            pltpu.VMEM((ROWS_PER_W,), jnp.int32),
            pltpu.VMEM((NSTEP, STEP, D), jnp.float32),
            pltpu.SemaphoreType.DMA((NSTEP,)),
            pltpu.SemaphoreType.DMA((NSTEP,)),
        ],
    )(_gather_kernel)(idx, table)


_ADD_BLOCK = 512


def _tc_add(tmp, pos):
    return pl.pallas_call(
        _add_kernel,
        grid=(SEQ // _ADD_BLOCK,),
        in_specs=[
            pl.BlockSpec((_ADD_BLOCK, D), lambda i: (i, 0)),
            pl.BlockSpec((_ADD_BLOCK, D), lambda i: (i, 0)),
        ],
        out_specs=pl.BlockSpec((_ADD_BLOCK, D), lambda i: (i, 0)),
        out_shape=jax.ShapeDtypeStruct((SEQ, D), jnp.float32),
    )(tmp, pos)


@jax.jit
def kernel(x, table, pos_encoding):
    idx = x.astype(jnp.int32)
    outs = []
    for b in range(BATCH):
        tmp = _sc_gather(idx[b], table)
        outs.append(_tc_add(tmp, pos_encoding))
    return jnp.stack(outs)
